# baseline (speedup 1.0000x reference)
import functools
import jax
import jax.numpy as jnp
from jax.experimental import pallas as pl
from jax.experimental.pallas import tpu as pltpu

N0, N1, N2 = 50000, 25000, 12500
NP0, NP1, NP2 = 51200, 25600, 12800
B = 64
KS1 = 8
R = 512


def _edge_plan(pseudo):
    f0 = pseudo[:, 0]
    v1 = pseudo[:, 1] * KS1
    i1 = jnp.floor(v1).astype(jnp.int32)
    f1 = v1 - i1
    i1a = jnp.mod(i1, KS1)
    i1b = jnp.mod(i1 + 1, KS1)
    w = jnp.stack([(1 - f0) * (1 - f1), (1 - f0) * f1,
                   f0 * (1 - f1), f0 * f1], axis=1)
    k = jnp.stack([i1a, i1b, KS1 + i1a, KS1 + i1b], axis=1)
    return k, w


def _scatter_S(x, edge_index, pseudo, num_pad):
    dst, src = edge_index[0], edge_index[1]
    k, w = _edge_plan(pseudo)
    xs = x[src]
    S = jnp.zeros((num_pad * 16, x.shape[1]), jnp.float32)
    for c in range(4):
        S = S.at[dst * 16 + k[:, c]].add(w[:, c:c + 1] * xs)
    deg = jnp.zeros((num_pad,), jnp.float32).at[dst].add(1.0)
    return S.reshape(num_pad, 16 * x.shape[1]), deg


def _dense_body(s_ref, x_ref, deg_ref, wf_ref, root_ref, b_ref, o_ref):
    agg = jnp.dot(s_ref[...], wf_ref[...], preferred_element_type=jnp.float32)
    inv = 1.0 / jnp.maximum(deg_ref[...], 1.0)
    h = agg * inv + jnp.dot(x_ref[...], root_ref[...],
                            preferred_element_type=jnp.float32) + b_ref[...]
    o_ref[...] = jnp.maximum(h, 0.0)


def _dense_layer(S, x, deg, W, root, bias):
    n = S.shape[0]
    cin = x.shape[1]
    cout = W.shape[2]
    wf = W.reshape(16 * cin, cout)
    grid = n // R
    return pl.pallas_call(
        _dense_body,
        grid=(grid,),
        in_specs=[
            pl.BlockSpec((R, 16 * cin), lambda r: (r, 0)),
            pl.BlockSpec((R, cin), lambda r: (r, 0)),
            pl.BlockSpec((R, 1), lambda r: (r, 0)),
            pl.BlockSpec((16 * cin, cout), lambda r: (0, 0)),
            pl.BlockSpec((cin, cout), lambda r: (0, 0)),
            pl.BlockSpec((1, cout), lambda r: (0, 0)),
        ],
        out_specs=pl.BlockSpec((R, cout), lambda r: (r, 0)),
        out_shape=jax.ShapeDtypeStruct((n, cout), jnp.float32),
    )(S, x, deg.reshape(n, 1), wf, root, bias.reshape(1, cout))


def _pool_body(h_ref, o_ref):
    o_ref[...] = jnp.maximum(h_ref[:, 0, :], h_ref[:, 1, :])


def _pool(h):
    n, c = h.shape
    h3 = h.reshape(n // 2, 2, c)
    grid = n // (2 * R)
    return pl.pallas_call(
        _pool_body,
        grid=(grid,),
        in_specs=[pl.BlockSpec((R, 2, c), lambda r: (r, 0, 0))],
        out_specs=pl.BlockSpec((R, c), lambda r: (r, 0)),
        out_shape=jax.ShapeDtypeStruct((n // 2, c), jnp.float32),
    )(h3)


def _readout_body(h_ref, sl_ref, fw_ref, fb_ref, o_ref, s_ref, c_ref):
    r = pl.program_id(0)
    nblk = pl.num_programs(0)

    @pl.when(r == 0)
    def _():
        s_ref[...] = jnp.zeros_like(s_ref)
        c_ref[...] = jnp.zeros_like(c_ref)

    rows = r * R + jax.lax.broadcasted_iota(jnp.int32, (R, 1), 0)
    cnt_le = jnp.sum((sl_ref[...] <= rows).astype(jnp.int32),
                     axis=1, keepdims=True)
    seg = jnp.clip(cnt_le, 0, B - 1)
    valid = (rows < N2).astype(jnp.float32)
    onehot = (seg == jax.lax.broadcasted_iota(jnp.int32, (R, B), 1)
              ).astype(jnp.float32) * valid
    s_ref[...] += jnp.dot(onehot.T, h_ref[...],
                          preferred_element_type=jnp.float32)
    c_ref[...] += jnp.sum(onehot, axis=0, keepdims=True)

    @pl.when(r == nblk - 1)
    def _():
        mean = s_ref[...] / jnp.maximum(c_ref[...], 1.0).T
        logits = jnp.dot(mean, fw_ref[...],
                         preferred_element_type=jnp.float32) + fb_ref[...]
        m = jnp.max(logits, axis=1, keepdims=True)
        lse = m + jnp.log(jnp.sum(jnp.exp(logits - m), axis=1, keepdims=True))
        o_ref[...] = logits - lse


def _readout(h, slice_, fc_w, fc_b):
    n = h.shape[0]
    grid = n // R
    out, _, _ = pl.pallas_call(
        _readout_body,
        grid=(grid,),
        in_specs=[
            pl.BlockSpec((R, 512), lambda r: (r, 0)),
            pl.BlockSpec((1, B), lambda r: (0, 0)),
            pl.BlockSpec((512, 10), lambda r: (0, 0)),
            pl.BlockSpec((1, 10), lambda r: (0, 0)),
        ],
        out_specs=[
            pl.BlockSpec((B, 10), lambda r: (0, 0)),
            pl.BlockSpec((B, 512), lambda r: (0, 0)),
            pl.BlockSpec((1, B), lambda r: (0, 0)),
        ],
        out_shape=[
            jax.ShapeDtypeStruct((B, 10), jnp.float32),
            jax.ShapeDtypeStruct((B, 512), jnp.float32),
            jax.ShapeDtypeStruct((1, B), jnp.float32),
        ],
    )(h, slice_.reshape(1, B).astype(jnp.int32), fc_w,
      fc_b.reshape(1, 10))
    return out


def kernel(x, edge_index0, pseudo0, edge_index1, pseudo1, edge_index2,
           pseudo2, slice_, W1, root1, b1, W2, root2, b2, W3, root3, b3,
           fc_w, fc_b):
    x0 = jnp.zeros((NP0, 2), jnp.float32).at[:N0].set(x)
    S0, deg0 = _scatter_S(x0[:N0], edge_index0, pseudo0, NP0)
    h = _dense_layer(S0, x0, deg0, W1, root1, b1)
    h = _pool(h)

    S1, deg1 = _scatter_S(h[:N1], edge_index1, pseudo1, NP1)
    h = _dense_layer(S1, h, deg1, W2, root2, b2)
    h = _pool(h)

    S2, deg2 = _scatter_S(h[:N2], edge_index2, pseudo2, NP2)
    h = _dense_layer(S2, h, deg2, W3, root3, b3)

    return _readout(h, slice_, fc_w, fc_b)


# speedup vs baseline: 2.9050x; 2.9050x over previous
import functools
import jax
import jax.numpy as jnp
from jax import lax
from jax.experimental import pallas as pl
from jax.experimental.pallas import tpu as pltpu
from jax.experimental.pallas import tpu_sc as plsc

N0, N1, N2 = 50000, 25000, 12500
NP0, NP1, NP2 = 51200, 25600, 12800
B = 64
KS1 = 8
R = 512
EW = 256
NB = 32


def _log2(v):
    return v.bit_length() - 1


def _sc_scatter(xp, dst, src, u0, u1, np_, cin, c, e_real):
    P = cin // c
    ph = max(P // 2, 1)
    n16 = np_ * 16
    sw = n16 * c
    swt = sw // 16
    zw = swt // 64
    brw = swt // NB
    e_pad = dst.shape[0]
    chunk = e_pad // 16
    nwin = chunk // EW
    xrt = np_ // 16
    mesh = plsc.VectorSubcoreMesh(core_axis_name="c", subcore_axis_name="s")

    @functools.partial(
        pl.kernel, mesh=mesh,
        compiler_params=pltpu.CompilerParams(
            needs_layout_passes=False, use_tc_tiling_on_sc=False),
        out_type=[jax.ShapeDtypeStruct((P, sw), jnp.float32),
                  jax.ShapeDtypeStruct((np_,), jnp.float32)],
        scratch_types=[
            pltpu.VMEM((np_ * c,), jnp.float32),
            pltpu.VMEM((EW,), jnp.int32),
            pltpu.VMEM((EW,), jnp.int32),
            pltpu.VMEM((EW,), jnp.float32),
            pltpu.VMEM((EW,), jnp.float32),
            pltpu.VMEM((4 * EW * c,), jnp.float32),
            pltpu.VMEM((4 * EW * c,), jnp.int32),
            pltpu.VMEM((EW,), jnp.float32),
            pltpu.VMEM((brw,), jnp.float32),
            pltpu.VMEM((zw,), jnp.float32),
            pltpu.VMEM((xrt,), jnp.float32),
            pltpu.VMEM_SHARED((sw,), jnp.float32),
            pltpu.VMEM_SHARED((np_ * c,), jnp.float32),
            pltpu.VMEM_SHARED((np_,), jnp.float32),
        ],
    )
    def scat(xp_hbm, dst_hbm, src_hbm, u0_hbm, u1_hbm, s_out, deg_out,
             xbuf, dstw, srcw, u0w, u1w, pay, sidx, degval, sbounce, z2,
             degz, s_sh, x_sh, deg_sh):
        ci = lax.axis_index("c")
        si = lax.axis_index("s")
        base_e = si * chunk
        iota = lax.iota(jnp.int32, 16)
        zf = jnp.zeros((16,), jnp.float32)

        def zero_vmem(ref, n):
            def zb(i, _):
                ref[pl.ds(i * 16, 16)] = zf
                return 0
            lax.fori_loop(0, n // 16, zb, 0)

        zero_vmem(z2, zw)

        def do_pass(p, _):
            q = ci * ph + p
            dodeg = jnp.logical_and(p == 0, ci == 0)
            def zs(b, _):
                pltpu.sync_copy(z2, s_sh.at[pl.ds(si * swt + b * zw, zw)])
                return 0
            lax.fori_loop(0, 64, zs, 0)

            @pl.when(dodeg)
            def _():
                zero_vmem(degz, xrt)
                pltpu.sync_copy(degz, deg_sh.at[pl.ds(si * xrt, xrt)])

            pltpu.sync_copy(xp_hbm.at[q, pl.ds(si * xrt * c, xrt * c)],
                            xbuf.at[pl.ds(0, xrt * c)])
            pltpu.sync_copy(xbuf.at[pl.ds(0, xrt * c)],
                            x_sh.at[pl.ds(si * xrt * c, xrt * c)])
            plsc.subcore_barrier()
            pltpu.sync_copy(x_sh, xbuf)

            def do_win(wn, _):
                be = base_e + wn * EW
                pltpu.sync_copy(dst_hbm.at[pl.ds(be, EW)], dstw)
                pltpu.sync_copy(src_hbm.at[pl.ds(be, EW)], srcw)
                pltpu.sync_copy(u0_hbm.at[pl.ds(be, EW)], u0w)
                pltpu.sync_copy(u1_hbm.at[pl.ds(be, EW)], u1w)
                for i in range(EW // 16):
                    glob = (be + i * 16) + iota
                    degval[pl.ds(i * 16, 16)] = jnp.where(
                        glob < e_real, 1.0, 0.0)
                epv = 16 // c
                ediv = lax.shift_right_logical(iota, _log2(c))
                jlane = iota & (c - 1)
                qn = EW * c // 16
                for i2 in range(qn):
                    e_ids = ediv + (i2 * epv)
                    vsr = plsc.load_gather(srcw, [e_ids])
                    vdr = plsc.load_gather(dstw, [e_ids])
                    u0r = plsc.load_gather(u0w, [e_ids])
                    u1r = plsc.load_gather(u1w, [e_ids])
                    xj = plsc.load_gather(xbuf, [vsr * c + jlane])
                    v1 = u1r * float(KS1)
                    i1 = v1.astype(jnp.int32)
                    f1 = v1 - i1.astype(jnp.float32)
                    i1b = (i1 + 1) & (KS1 - 1)
                    b16c = vdr * (16 * c) + jlane
                    s0 = b16c + i1 * c
                    s1 = b16c + i1b * c
                    globr = (be + i2 * epv) + ediv
                    mf = jnp.where(globr < e_real, 1.0, 0.0)
                    f0 = u0r * mf
                    g0 = mf - f0
                    xg1 = xj * (1.0 - f1)
                    xf1 = xj * f1
                    pay[pl.ds((0 * qn + i2) * 16, 16)] = g0 * xg1
                    pay[pl.ds((1 * qn + i2) * 16, 16)] = g0 * xf1
                    pay[pl.ds((2 * qn + i2) * 16, 16)] = f0 * xg1
                    pay[pl.ds((3 * qn + i2) * 16, 16)] = f0 * xf1
                    sidx[pl.ds((0 * qn + i2) * 16, 16)] = s0
                    sidx[pl.ds((1 * qn + i2) * 16, 16)] = s1
                    sidx[pl.ds((2 * qn + i2) * 16, 16)] = s0 + 8 * c
                    sidx[pl.ds((3 * qn + i2) * 16, 16)] = s1 + 8 * c
                pltpu.sync_copy(pay, s_sh.at[sidx], add=True)

                @pl.when(dodeg)
                def _():
                    pltpu.sync_copy(degval, deg_sh.at[dstw], add=True)
                return 0
            lax.fori_loop(0, nwin, do_win, 0)
            plsc.subcore_barrier()

            def dr(b, _):
                ro = si * swt + b * brw
                pltpu.sync_copy(s_sh.at[pl.ds(ro, brw)], sbounce)
                pltpu.sync_copy(sbounce, s_out.at[q, pl.ds(ro, brw)])
                return 0
            lax.fori_loop(0, NB, dr, 0)

            @pl.when(dodeg)
            def _():
                pltpu.sync_copy(deg_sh.at[pl.ds(si * xrt, xrt)], degz)
                pltpu.sync_copy(degz, deg_out.at[pl.ds(si * xrt, xrt)])
            plsc.subcore_barrier()
            return 0

        lax.fori_loop(0, ph, do_pass, 0)

    return scat(xp, dst, src, u0, u1)


def _prep_edges(edge_index, pseudo, e_pad, np_):
    e = edge_index.shape[1]
    pad = e_pad - e
    spread = (jnp.arange(pad, dtype=jnp.int32) % np_)
    z = jnp.zeros((pad,), jnp.float32)
    dst = jnp.concatenate([edge_index[0].astype(jnp.int32), spread])
    src = jnp.concatenate([edge_index[1].astype(jnp.int32),
                           jnp.zeros((pad,), jnp.int32)])
    u0 = jnp.concatenate([pseudo[:, 0], z])
    u1 = jnp.concatenate([pseudo[:, 1], z])
    return dst, src, u0, u1


def _scatter_S(x_pad, edge_index, pseudo, np_, c):
    cin = x_pad.shape[1]
    P = cin // c
    e = edge_index.shape[1]
    e_pad = ((e + 16 * EW - 1) // (16 * EW)) * (16 * EW)
    dst, src, u0, u1 = _prep_edges(edge_index, pseudo, e_pad, np_)
    xp = x_pad.reshape(np_, P, c).transpose(1, 0, 2).reshape(P, np_ * c)
    S, deg = _sc_scatter(xp, dst, src, u0, u1, np_, cin, c, e)
    return S.reshape(P, np_, 16 * c), deg


def _dense_body(s_ref, x_ref, deg_ref, wf_ref, root_ref, b_ref, o_ref, *,
                pp):
    agg = jnp.dot(s_ref[0], wf_ref[0], preferred_element_type=jnp.float32)
    for q in range(1, pp):
        agg += jnp.dot(s_ref[q], wf_ref[q],
                       preferred_element_type=jnp.float32)
    inv = 1.0 / jnp.maximum(deg_ref[...], 1.0)
    h = agg * inv + jnp.dot(x_ref[...], root_ref[...],
                            preferred_element_type=jnp.float32) + b_ref[...]
    o_ref[...] = jnp.maximum(h, 0.0)


def _dense_layer(S, x, deg, W, root, bias, c):
    n = x.shape[0]
    cin = x.shape[1]
    cout = W.shape[2]
    pp = cin // c
    wf = W.reshape(16, pp, c, cout).transpose(1, 0, 2, 3).reshape(
        pp, 16 * c, cout)
    grid = n // R
    return pl.pallas_call(
        functools.partial(_dense_body, pp=pp),
        grid=(grid,),
        in_specs=[
            pl.BlockSpec((pp, R, 16 * c), lambda r: (0, r, 0)),
            pl.BlockSpec((R, cin), lambda r: (r, 0)),
            pl.BlockSpec((R, 1), lambda r: (r, 0)),
            pl.BlockSpec((pp, 16 * c, cout), lambda r: (0, 0, 0)),
            pl.BlockSpec((cin, cout), lambda r: (0, 0)),
            pl.BlockSpec((1, cout), lambda r: (0, 0)),
        ],
        out_specs=pl.BlockSpec((R, cout), lambda r: (r, 0)),
        out_shape=jax.ShapeDtypeStruct((n, cout), jnp.float32),
    )(S, x, deg.reshape(n, 1), wf, root, bias.reshape(1, cout))


def _pool_body(h_ref, o_ref):
    o_ref[...] = jnp.maximum(h_ref[:, 0, :], h_ref[:, 1, :])


def _pool(h):
    n, c = h.shape
    h3 = h.reshape(n // 2, 2, c)
    grid = n // (2 * R)
    return pl.pallas_call(
        _pool_body,
        grid=(grid,),
        in_specs=[pl.BlockSpec((R, 2, c), lambda r: (r, 0, 0))],
        out_specs=pl.BlockSpec((R, c), lambda r: (r, 0)),
        out_shape=jax.ShapeDtypeStruct((n // 2, c), jnp.float32),
    )(h3)


def _readout_body(h_ref, sl_ref, fw_ref, fb_ref, o_ref, s_ref, c_ref):
    r = pl.program_id(0)
    nblk = pl.num_programs(0)

    @pl.when(r == 0)
    def _():
        s_ref[...] = jnp.zeros_like(s_ref)
        c_ref[...] = jnp.zeros_like(c_ref)

    rows = r * R + jax.lax.broadcasted_iota(jnp.int32, (R, 1), 0)
    cnt_le = jnp.sum((sl_ref[...] <= rows).astype(jnp.int32),
                     axis=1, keepdims=True)
    seg = jnp.clip(cnt_le, 0, B - 1)
    valid = (rows < N2).astype(jnp.float32)
    onehot = (seg == jax.lax.broadcasted_iota(jnp.int32, (R, B), 1)
              ).astype(jnp.float32) * valid
    s_ref[...] += jnp.dot(onehot.T, h_ref[...],
                          preferred_element_type=jnp.float32)
    c_ref[...] += jnp.sum(onehot, axis=0, keepdims=True)

    @pl.when(r == nblk - 1)
    def _():
        mean = s_ref[...] / jnp.maximum(c_ref[...], 1.0).T
        logits = jnp.dot(mean, fw_ref[...],
                         preferred_element_type=jnp.float32) + fb_ref[...]
        m = jnp.max(logits, axis=1, keepdims=True)
        lse = m + jnp.log(jnp.sum(jnp.exp(logits - m), axis=1, keepdims=True))
        o_ref[...] = logits - lse


def _readout(h, slice_, fc_w, fc_b):
    n = h.shape[0]
    grid = n // R
    out, _, _ = pl.pallas_call(
        _readout_body,
        grid=(grid,),
        in_specs=[
            pl.BlockSpec((R, 512), lambda r: (r, 0)),
            pl.BlockSpec((1, B), lambda r: (0, 0)),
            pl.BlockSpec((512, 10), lambda r: (0, 0)),
            pl.BlockSpec((1, 10), lambda r: (0, 0)),
        ],
        out_specs=[
            pl.BlockSpec((B, 10), lambda r: (0, 0)),
            pl.BlockSpec((B, 512), lambda r: (0, 0)),
            pl.BlockSpec((1, B), lambda r: (0, 0)),
        ],
        out_shape=[
            jax.ShapeDtypeStruct((B, 10), jnp.float32),
            jax.ShapeDtypeStruct((B, 512), jnp.float32),
            jax.ShapeDtypeStruct((1, B), jnp.float32),
        ],
    )(h, slice_.reshape(1, B).astype(jnp.int32), fc_w,
      fc_b.reshape(1, 10))
    return out


def kernel(x, edge_index0, pseudo0, edge_index1, pseudo1, edge_index2,
           pseudo2, slice_, W1, root1, b1, W2, root2, b2, W3, root3, b3,
           fc_w, fc_b):
    x0 = jnp.zeros((NP0, 2), jnp.float32).at[:N0].set(x)
    S0, deg0 = _scatter_S(x0, edge_index0, pseudo0, NP0, 1)
    h = _dense_layer(S0, x0, deg0, W1, root1, b1, 1)
    h = _pool(h)

    S1, deg1 = _scatter_S(h, edge_index1, pseudo1, NP1, 2)
    h = _dense_layer(S1, h, deg1, W2, root2, b2, 2)
    h = _pool(h)

    S2, deg2 = _scatter_S(h, edge_index2, pseudo2, NP2, 4)
    h = _dense_layer(S2, h, deg2, W3, root3, b3, 4)

    return _readout(h, slice_, fc_w, fc_b)


# speedup vs baseline: 3.1270x; 1.0764x over previous
import functools
import jax
import jax.numpy as jnp
from jax import lax
from jax.experimental import pallas as pl
from jax.experimental.pallas import tpu as pltpu
from jax.experimental.pallas import tpu_sc as plsc

N0, N1, N2 = 50000, 25000, 12500
NP0, NP1, NP2 = 51200, 25600, 12800
B = 64
KS1 = 8
R = 512
EW = 256
NB = 32


def _log2(v):
    return v.bit_length() - 1


def _sc_scatter(xp, ei4, np_, cin, c, e_real):
    P = cin // c
    ph = max(P // 2, 1)
    n16 = np_ * 16
    sw = n16 * c
    swt = sw // 16
    zw = swt // 64
    brw = swt // NB
    e_pad = ei4.shape[0] // 4
    chunk = e_pad // 16
    nwin = chunk // EW
    xrt = np_ // 16
    mesh = plsc.VectorSubcoreMesh(core_axis_name="c", subcore_axis_name="s")

    @functools.partial(
        pl.kernel, mesh=mesh,
        compiler_params=pltpu.CompilerParams(
            needs_layout_passes=False, use_tc_tiling_on_sc=False),
        out_type=[jax.ShapeDtypeStruct((P, sw), jnp.float32),
                  jax.ShapeDtypeStruct((np_,), jnp.float32)],
        scratch_types=[
            pltpu.VMEM((np_ * c,), jnp.float32),
            pltpu.VMEM((4 * EW,), jnp.int32),
            pltpu.VMEM((4 * EW,), jnp.int32),
            pltpu.VMEM((EW,), jnp.int32),
            pltpu.VMEM((4 * EW * c,), jnp.float32),
            pltpu.VMEM((4 * EW * c,), jnp.int32),
            pltpu.VMEM((EW,), jnp.float32),
            pltpu.VMEM((brw,), jnp.float32),
            pltpu.VMEM((zw,), jnp.float32),
            pltpu.VMEM((xrt,), jnp.float32),
            pltpu.SemaphoreType.DMA,
            pltpu.VMEM_SHARED((sw,), jnp.float32),
            pltpu.VMEM_SHARED((np_,), jnp.float32),
        ],
    )
    def scat(xp_hbm, ei_hbm, s_out, deg_out,
             xbuf, ebufa, ebufb, dstw, pay, sidx, degval, sbounce, z2,
             degz, esem, s_sh, deg_sh):
        ci = lax.axis_index("c")
        si = lax.axis_index("s")
        base_e = si * chunk
        iota = lax.iota(jnp.int32, 16)
        zf = jnp.zeros((16,), jnp.float32)

        def zero_vmem(ref, n):
            def zb(i, _):
                ref[pl.ds(i * 16, 16)] = zf
                return 0
            lax.fori_loop(0, n // 16, zb, 0)

        zero_vmem(z2, zw)

        def issue_edges(w, eb):
            pltpu.async_copy(
                ei_hbm.at[pl.ds((base_e + w * EW) * 4, EW * 4)], eb, esem)

        def wait_edges(eb):
            pltpu.make_async_copy(
                ei_hbm.at[pl.ds(0, EW * 4)], eb, esem).wait()

        def do_pass(p, _):
            q = ci * ph + p
            dodeg = jnp.logical_and(p == 0, ci == 0)
            def zs(b, _):
                pltpu.sync_copy(z2, s_sh.at[pl.ds(si * swt + b * zw, zw)])
                return 0
            lax.fori_loop(0, 64, zs, 0)

            @pl.when(dodeg)
            def _():
                zero_vmem(degz, xrt)
                pltpu.sync_copy(degz, deg_sh.at[pl.ds(si * xrt, xrt)])

            pltpu.sync_copy(xp_hbm.at[q], xbuf)
            plsc.subcore_barrier()

            epv = 16 // c
            ediv = lax.shift_right_logical(iota, _log2(c))
            jlane = iota & (c - 1)
            qn = EW * c // 16

            def do_win(wn, eb):
                be = base_e + wn * EW

                @pl.when(dodeg)
                def _():
                    for i in range(EW // 16):
                        eidx = (i * 16) + iota
                        vd = plsc.load_gather(eb, [eidx * 4])
                        glob = be + eidx
                        dstw[pl.ds(i * 16, 16)] = vd
                        degval[pl.ds(i * 16, 16)] = jnp.where(
                            glob < e_real, 1.0, 0.0)

                for i2 in range(qn):
                    e_ids = ediv + (i2 * epv)
                    e4 = e_ids * 4
                    vdr = plsc.load_gather(eb, [e4])
                    vsr = plsc.load_gather(eb, [e4 + 1])
                    u0r = plsc.bitcast(plsc.load_gather(eb, [e4 + 2]),
                                       jnp.float32)
                    u1r = plsc.bitcast(plsc.load_gather(eb, [e4 + 3]),
                                       jnp.float32)
                    xj = plsc.load_gather(xbuf, [vsr * c + jlane])
                    v1 = u1r * float(KS1)
                    i1 = v1.astype(jnp.int32)
                    f1 = v1 - i1.astype(jnp.float32)
                    i1b = (i1 + 1) & (KS1 - 1)
                    b16c = vdr * (16 * c) + jlane
                    s0 = b16c + i1 * c
                    s1 = b16c + i1b * c
                    globr = (be + i2 * epv) + ediv
                    mf = jnp.where(globr < e_real, 1.0, 0.0)
                    f0 = u0r * mf
                    g0 = mf - f0
                    xg1 = xj * (1.0 - f1)
                    xf1 = xj * f1
                    pay[pl.ds((0 * qn + i2) * 16, 16)] = g0 * xg1
                    pay[pl.ds((1 * qn + i2) * 16, 16)] = g0 * xf1
                    pay[pl.ds((2 * qn + i2) * 16, 16)] = f0 * xg1
                    pay[pl.ds((3 * qn + i2) * 16, 16)] = f0 * xf1
                    sidx[pl.ds((0 * qn + i2) * 16, 16)] = s0
                    sidx[pl.ds((1 * qn + i2) * 16, 16)] = s1
                    sidx[pl.ds((2 * qn + i2) * 16, 16)] = s0 + 8 * c
                    sidx[pl.ds((3 * qn + i2) * 16, 16)] = s1 + 8 * c
                pltpu.sync_copy(pay, s_sh.at[sidx], add=True)

                @pl.when(dodeg)
                def _():
                    pltpu.sync_copy(degval, deg_sh.at[dstw], add=True)

            issue_edges(0, ebufa)

            def wpair(t, _):
                w0 = 2 * t
                wait_edges(ebufa)
                issue_edges(w0 + 1, ebufb)
                do_win(w0, ebufa)
                wait_edges(ebufb)

                @pl.when(w0 + 2 < nwin)
                def _():
                    issue_edges(w0 + 2, ebufa)
                do_win(w0 + 1, ebufb)
                return 0
            lax.fori_loop(0, nwin // 2, wpair, 0)
            plsc.subcore_barrier()

            def dr(b, _):
                ro = si * swt + b * brw
                pltpu.sync_copy(s_sh.at[pl.ds(ro, brw)], sbounce)
                pltpu.sync_copy(sbounce, s_out.at[q, pl.ds(ro, brw)])
                return 0
            lax.fori_loop(0, NB, dr, 0)

            @pl.when(dodeg)
            def _():
                pltpu.sync_copy(deg_sh.at[pl.ds(si * xrt, xrt)], degz)
                pltpu.sync_copy(degz, deg_out.at[pl.ds(si * xrt, xrt)])
            plsc.subcore_barrier()
            return 0

        lax.fori_loop(0, ph, do_pass, 0)

    return scat(xp, ei4)


def _prep_edges(edge_index, pseudo, e_pad, np_):
    e = edge_index.shape[1]
    pad = e_pad - e
    spread = (jnp.arange(pad, dtype=jnp.int32) % np_)
    zi = jnp.zeros((pad,), jnp.int32)
    dst = jnp.concatenate([edge_index[0].astype(jnp.int32), spread])
    src = jnp.concatenate([edge_index[1].astype(jnp.int32), zi])
    u0 = jnp.concatenate([lax.bitcast_convert_type(pseudo[:, 0], jnp.int32),
                          zi])
    u1 = jnp.concatenate([lax.bitcast_convert_type(pseudo[:, 1], jnp.int32),
                          zi])
    return jnp.stack([dst, src, u0, u1], axis=1).reshape(-1)


def _scatter_S(x_pad, edge_index, pseudo, np_, c):
    cin = x_pad.shape[1]
    P = cin // c
    e = edge_index.shape[1]
    e_pad = ((e + 32 * EW - 1) // (32 * EW)) * (32 * EW)
    ei4 = _prep_edges(edge_index, pseudo, e_pad, np_)
    xp = x_pad.reshape(np_, P, c).transpose(1, 0, 2).reshape(P, np_ * c)
    S, deg = _sc_scatter(xp, ei4, np_, cin, c, e)
    return S.reshape(P, np_, 16 * c), deg


def _dense_body(s_ref, x_ref, deg_ref, wf_ref, root_ref, b_ref, o_ref, *,
                pp):
    agg = jnp.dot(s_ref[0], wf_ref[0], preferred_element_type=jnp.float32)
    for q in range(1, pp):
        agg += jnp.dot(s_ref[q], wf_ref[q],
                       preferred_element_type=jnp.float32)
    inv = 1.0 / jnp.maximum(deg_ref[...], 1.0)
    h = agg * inv + jnp.dot(x_ref[...], root_ref[...],
                            preferred_element_type=jnp.float32) + b_ref[...]
    o_ref[...] = jnp.maximum(h, 0.0)


def _dense_layer(S, x, deg, W, root, bias, c):
    n = x.shape[0]
    cin = x.shape[1]
    cout = W.shape[2]
    pp = cin // c
    wf = W.reshape(16, pp, c, cout).transpose(1, 0, 2, 3).reshape(
        pp, 16 * c, cout)
    grid = n // R
    return pl.pallas_call(
        functools.partial(_dense_body, pp=pp),
        grid=(grid,),
        in_specs=[
            pl.BlockSpec((pp, R, 16 * c), lambda r: (0, r, 0)),
            pl.BlockSpec((R, cin), lambda r: (r, 0)),
            pl.BlockSpec((R, 1), lambda r: (r, 0)),
            pl.BlockSpec((pp, 16 * c, cout), lambda r: (0, 0, 0)),
            pl.BlockSpec((cin, cout), lambda r: (0, 0)),
            pl.BlockSpec((1, cout), lambda r: (0, 0)),
        ],
        out_specs=pl.BlockSpec((R, cout), lambda r: (r, 0)),
        out_shape=jax.ShapeDtypeStruct((n, cout), jnp.float32),
    )(S, x, deg.reshape(n, 1), wf, root, bias.reshape(1, cout))


def _pool_body(h_ref, o_ref):
    o_ref[...] = jnp.maximum(h_ref[:, 0, :], h_ref[:, 1, :])


def _pool(h):
    n, c = h.shape
    h3 = h.reshape(n // 2, 2, c)
    grid = n // (2 * R)
    return pl.pallas_call(
        _pool_body,
        grid=(grid,),
        in_specs=[pl.BlockSpec((R, 2, c), lambda r: (r, 0, 0))],
        out_specs=pl.BlockSpec((R, c), lambda r: (r, 0)),
        out_shape=jax.ShapeDtypeStruct((n // 2, c), jnp.float32),
    )(h3)


def _readout_body(h_ref, sl_ref, fw_ref, fb_ref, o_ref, s_ref, c_ref):
    r = pl.program_id(0)
    nblk = pl.num_programs(0)

    @pl.when(r == 0)
    def _():
        s_ref[...] = jnp.zeros_like(s_ref)
        c_ref[...] = jnp.zeros_like(c_ref)

    rows = r * R + jax.lax.broadcasted_iota(jnp.int32, (R, 1), 0)
    cnt_le = jnp.sum((sl_ref[...] <= rows).astype(jnp.int32),
                     axis=1, keepdims=True)
    seg = jnp.clip(cnt_le, 0, B - 1)
    valid = (rows < N2).astype(jnp.float32)
    onehot = (seg == jax.lax.broadcasted_iota(jnp.int32, (R, B), 1)
              ).astype(jnp.float32) * valid
    s_ref[...] += jnp.dot(onehot.T, h_ref[...],
                          preferred_element_type=jnp.float32)
    c_ref[...] += jnp.sum(onehot, axis=0, keepdims=True)

    @pl.when(r == nblk - 1)
    def _():
        mean = s_ref[...] / jnp.maximum(c_ref[...], 1.0).T
        logits = jnp.dot(mean, fw_ref[...],
                         preferred_element_type=jnp.float32) + fb_ref[...]
        m = jnp.max(logits, axis=1, keepdims=True)
        lse = m + jnp.log(jnp.sum(jnp.exp(logits - m), axis=1, keepdims=True))
        o_ref[...] = logits - lse


def _readout(h, slice_, fc_w, fc_b):
    n = h.shape[0]
    grid = n // R
    out, _, _ = pl.pallas_call(
        _readout_body,
        grid=(grid,),
        in_specs=[
            pl.BlockSpec((R, 512), lambda r: (r, 0)),
            pl.BlockSpec((1, B), lambda r: (0, 0)),
            pl.BlockSpec((512, 10), lambda r: (0, 0)),
            pl.BlockSpec((1, 10), lambda r: (0, 0)),
        ],
        out_specs=[
            pl.BlockSpec((B, 10), lambda r: (0, 0)),
            pl.BlockSpec((B, 512), lambda r: (0, 0)),
            pl.BlockSpec((1, B), lambda r: (0, 0)),
        ],
        out_shape=[
            jax.ShapeDtypeStruct((B, 10), jnp.float32),
            jax.ShapeDtypeStruct((B, 512), jnp.float32),
            jax.ShapeDtypeStruct((1, B), jnp.float32),
        ],
    )(h, slice_.reshape(1, B).astype(jnp.int32), fc_w,
      fc_b.reshape(1, 10))
    return out


def kernel(x, edge_index0, pseudo0, edge_index1, pseudo1, edge_index2,
           pseudo2, slice_, W1, root1, b1, W2, root2, b2, W3, root3, b3,
           fc_w, fc_b):
    x0 = jnp.zeros((NP0, 2), jnp.float32).at[:N0].set(x)
    S0, deg0 = _scatter_S(x0, edge_index0, pseudo0, NP0, 1)
    h = _dense_layer(S0, x0, deg0, W1, root1, b1, 1)
    h = _pool(h)

    S1, deg1 = _scatter_S(h, edge_index1, pseudo1, NP1, 2)
    h = _dense_layer(S1, h, deg1, W2, root2, b2, 2)
    h = _pool(h)

    S2, deg2 = _scatter_S(h, edge_index2, pseudo2, NP2, 4)
    h = _dense_layer(S2, h, deg2, W3, root3, b3, 4)

    return _readout(h, slice_, fc_w, fc_b)


# speedup vs baseline: 3.1311x; 1.0013x over previous
import functools
import jax
import jax.numpy as jnp
from jax import lax
from jax.experimental import pallas as pl
from jax.experimental.pallas import tpu as pltpu
from jax.experimental.pallas import tpu_sc as plsc

N0, N1, N2 = 50000, 25000, 12500
NP0, NP1, NP2 = 51200, 25600, 12800
B = 64
KS1 = 8
R = 512
EW = 256
NB = 32


def _log2(v):
    return v.bit_length() - 1


def _sc_scatter(xp, ei4, np_, cin, c, e_real, ew):
    P = cin // c
    ph = max(P // 2, 1)
    n16 = np_ * 16
    sw = n16 * c
    swt = sw // 16
    zw = swt // 64
    brw = swt // NB
    e_pad = ei4.shape[0] // 4
    chunk = e_pad // 16
    nwin = chunk // ew
    xrt = np_ // 16
    mesh = plsc.VectorSubcoreMesh(core_axis_name="c", subcore_axis_name="s")

    @functools.partial(
        pl.kernel, mesh=mesh,
        compiler_params=pltpu.CompilerParams(
            needs_layout_passes=False, use_tc_tiling_on_sc=False),
        out_type=[jax.ShapeDtypeStruct((P, sw), jnp.float32),
                  jax.ShapeDtypeStruct((np_,), jnp.float32)],
        scratch_types=[
            pltpu.VMEM((np_ * c,), jnp.float32),
            pltpu.VMEM((4 * ew,), jnp.int32),
            pltpu.VMEM((4 * ew,), jnp.int32),
            pltpu.VMEM((ew,), jnp.int32),
            pltpu.VMEM((4 * ew * c,), jnp.float32),
            pltpu.VMEM((4 * ew * c,), jnp.float32),
            pltpu.VMEM((4 * ew * c,), jnp.int32),
            pltpu.VMEM((4 * ew * c,), jnp.int32),
            pltpu.VMEM((ew,), jnp.float32),
            pltpu.VMEM((brw,), jnp.float32),
            pltpu.VMEM((zw,), jnp.float32),
            pltpu.VMEM((xrt,), jnp.float32),
            pltpu.SemaphoreType.DMA,
            pltpu.SemaphoreType.DMA,
            pltpu.SemaphoreType.DMA,
            pltpu.VMEM_SHARED((sw,), jnp.float32),
            pltpu.VMEM_SHARED((np_,), jnp.float32),
        ],
    )
    def scat(xp_hbm, ei_hbm, s_out, deg_out,
             xbuf, ebufa, ebufb, dstw, paya, payb, sidxa, sidxb, degval,
             sbounce, z2, degz, esem, ssema, ssemb, s_sh, deg_sh):
        ci = lax.axis_index("c")
        si = lax.axis_index("s")
        base_e = si * chunk
        iota = lax.iota(jnp.int32, 16)
        zf = jnp.zeros((16,), jnp.float32)

        def zero_vmem(ref, n):
            def zb(i, _):
                ref[pl.ds(i * 16, 16)] = zf
                return 0
            lax.fori_loop(0, n // 16, zb, 0)

        zero_vmem(z2, zw)

        def issue_edges(w, eb):
            pltpu.async_copy(
                ei_hbm.at[pl.ds((base_e + w * ew) * 4, ew * 4)], eb, esem)

        def wait_edges(eb):
            pltpu.make_async_copy(
                ei_hbm.at[pl.ds(0, ew * 4)], eb, esem).wait()

        def issue_scatter(pay, sidx, sem):
            pltpu.async_copy(pay, s_sh.at[sidx], sem)

        def wait_scatter(pay, sidx, sem):
            pltpu.make_async_copy(pay, s_sh.at[sidx], sem).wait()

        def do_pass(p, _):
            q = ci * ph + p
            dodeg = jnp.logical_and(p == 0, ci == 0)
            def zs(b, _):
                pltpu.sync_copy(z2, s_sh.at[pl.ds(si * swt + b * zw, zw)])
                return 0
            lax.fori_loop(0, 64, zs, 0)

            @pl.when(dodeg)
            def _():
                zero_vmem(degz, xrt)
                pltpu.sync_copy(degz, deg_sh.at[pl.ds(si * xrt, xrt)])

            pltpu.sync_copy(xp_hbm.at[q], xbuf)
            plsc.subcore_barrier()

            epv = 16 // c
            ediv = lax.shift_right_logical(iota, _log2(c))
            jlane = iota & (c - 1)
            qn = ew * c // 16

            def do_win(wn, eb, pay, sidx, sem):
                be = base_e + wn * ew

                @pl.when(dodeg)
                def _():
                    for i in range(ew // 16):
                        eidx = (i * 16) + iota
                        vd = plsc.load_gather(eb, [eidx * 4])
                        glob = be + eidx
                        dstw[pl.ds(i * 16, 16)] = vd
                        degval[pl.ds(i * 16, 16)] = jnp.where(
                            glob < e_real, 1.0, 0.0)

                for i2 in range(qn):
                    e_ids = ediv + (i2 * epv)
                    e4 = e_ids * 4
                    vdr = plsc.load_gather(eb, [e4])
                    vsr = plsc.load_gather(eb, [e4 + 1])
                    u0r = plsc.bitcast(plsc.load_gather(eb, [e4 + 2]),
                                       jnp.float32)
                    u1r = plsc.bitcast(plsc.load_gather(eb, [e4 + 3]),
                                       jnp.float32)
                    xj = plsc.load_gather(xbuf, [vsr * c + jlane])
                    v1 = u1r * float(KS1)
                    i1 = v1.astype(jnp.int32)
                    f1 = v1 - i1.astype(jnp.float32)
                    i1b = (i1 + 1) & (KS1 - 1)
                    b16c = vdr * (16 * c) + jlane
                    s0 = b16c + i1 * c
                    s1 = b16c + i1b * c
                    globr = (be + i2 * epv) + ediv
                    mf = jnp.where(globr < e_real, 1.0, 0.0)
                    f0 = u0r * mf
                    g0 = mf - f0
                    xg1 = xj * (1.0 - f1)
                    xf1 = xj * f1
                    pay[pl.ds((0 * qn + i2) * 16, 16)] = g0 * xg1
                    pay[pl.ds((1 * qn + i2) * 16, 16)] = g0 * xf1
                    pay[pl.ds((2 * qn + i2) * 16, 16)] = f0 * xg1
                    pay[pl.ds((3 * qn + i2) * 16, 16)] = f0 * xf1
                    sidx[pl.ds((0 * qn + i2) * 16, 16)] = s0
                    sidx[pl.ds((1 * qn + i2) * 16, 16)] = s1
                    sidx[pl.ds((2 * qn + i2) * 16, 16)] = s0 + 8 * c
                    sidx[pl.ds((3 * qn + i2) * 16, 16)] = s1 + 8 * c
                pltpu.sync_copy(pay, s_sh.at[sidx], add=True)

                @pl.when(dodeg)
                def _():
                    pltpu.sync_copy(degval, deg_sh.at[dstw], add=True)

            issue_edges(0, ebufa)

            def wpair(t, _):
                w0 = 2 * t
                wait_edges(ebufa)
                issue_edges(w0 + 1, ebufb)
                do_win(w0, ebufa, paya, sidxa, ssema)
                wait_edges(ebufb)

                @pl.when(w0 + 2 < nwin)
                def _():
                    issue_edges(w0 + 2, ebufa)
                do_win(w0 + 1, ebufb, payb, sidxb, ssemb)
                return 0
            lax.fori_loop(0, nwin // 2, wpair, 0)
            plsc.subcore_barrier()

            def dr(b, _):
                ro = si * swt + b * brw
                pltpu.sync_copy(s_sh.at[pl.ds(ro, brw)], sbounce)
                pltpu.sync_copy(sbounce, s_out.at[q, pl.ds(ro, brw)])
                return 0
            lax.fori_loop(0, NB, dr, 0)

            @pl.when(dodeg)
            def _():
                pltpu.sync_copy(deg_sh.at[pl.ds(si * xrt, xrt)], degz)
                pltpu.sync_copy(degz, deg_out.at[pl.ds(si * xrt, xrt)])
            plsc.subcore_barrier()
            return 0

        lax.fori_loop(0, ph, do_pass, 0)

    return scat(xp, ei4)


def _prep_edges(edge_index, pseudo, e_pad, np_):
    e = edge_index.shape[1]
    pad = e_pad - e
    spread = (jnp.arange(pad, dtype=jnp.int32) % np_)
    zi = jnp.zeros((pad,), jnp.int32)
    dst = jnp.concatenate([edge_index[0].astype(jnp.int32), spread])
    src = jnp.concatenate([edge_index[1].astype(jnp.int32), zi])
    u0 = jnp.concatenate([lax.bitcast_convert_type(pseudo[:, 0], jnp.int32),
                          zi])
    u1 = jnp.concatenate([lax.bitcast_convert_type(pseudo[:, 1], jnp.int32),
                          zi])
    return jnp.stack([dst, src, u0, u1], axis=1).reshape(-1)


def _scatter_S(x_pad, edge_index, pseudo, np_, c):
    cin = x_pad.shape[1]
    P = cin // c
    e = edge_index.shape[1]
    ew = 256
    e_pad = ((e + 32 * ew - 1) // (32 * ew)) * (32 * ew)
    ei4 = _prep_edges(edge_index, pseudo, e_pad, np_)
    xp = x_pad.reshape(np_, P, c).transpose(1, 0, 2).reshape(P, np_ * c)
    S, deg = _sc_scatter(xp, ei4, np_, cin, c, e, ew)
    return S.reshape(P, np_, 16 * c), deg


def _dense_body(s_ref, x_ref, deg_ref, wf_ref, root_ref, b_ref, o_ref, *,
                pp):
    agg = jnp.dot(s_ref[0], wf_ref[0], preferred_element_type=jnp.float32)
    for q in range(1, pp):
        agg += jnp.dot(s_ref[q], wf_ref[q],
                       preferred_element_type=jnp.float32)
    inv = 1.0 / jnp.maximum(deg_ref[...], 1.0)
    h = agg * inv + jnp.dot(x_ref[...], root_ref[...],
                            preferred_element_type=jnp.float32) + b_ref[...]
    o_ref[...] = jnp.maximum(h, 0.0)


def _dense_layer(S, x, deg, W, root, bias, c):
    n = x.shape[0]
    cin = x.shape[1]
    cout = W.shape[2]
    pp = cin // c
    wf = W.reshape(16, pp, c, cout).transpose(1, 0, 2, 3).reshape(
        pp, 16 * c, cout)
    grid = n // R
    return pl.pallas_call(
        functools.partial(_dense_body, pp=pp),
        grid=(grid,),
        in_specs=[
            pl.BlockSpec((pp, R, 16 * c), lambda r: (0, r, 0)),
            pl.BlockSpec((R, cin), lambda r: (r, 0)),
            pl.BlockSpec((R, 1), lambda r: (r, 0)),
            pl.BlockSpec((pp, 16 * c, cout), lambda r: (0, 0, 0)),
            pl.BlockSpec((cin, cout), lambda r: (0, 0)),
            pl.BlockSpec((1, cout), lambda r: (0, 0)),
        ],
        out_specs=pl.BlockSpec((R, cout), lambda r: (r, 0)),
        out_shape=jax.ShapeDtypeStruct((n, cout), jnp.float32),
    )(S, x, deg.reshape(n, 1), wf, root, bias.reshape(1, cout))


def _pool_body(h_ref, o_ref):
    o_ref[...] = jnp.maximum(h_ref[:, 0, :], h_ref[:, 1, :])


def _pool(h):
    n, c = h.shape
    h3 = h.reshape(n // 2, 2, c)
    grid = n // (2 * R)
    return pl.pallas_call(
        _pool_body,
        grid=(grid,),
        in_specs=[pl.BlockSpec((R, 2, c), lambda r: (r, 0, 0))],
        out_specs=pl.BlockSpec((R, c), lambda r: (r, 0)),
        out_shape=jax.ShapeDtypeStruct((n // 2, c), jnp.float32),
    )(h3)


def _readout_body(h_ref, sl_ref, fw_ref, fb_ref, o_ref, s_ref, c_ref):
    r = pl.program_id(0)
    nblk = pl.num_programs(0)

    @pl.when(r == 0)
    def _():
        s_ref[...] = jnp.zeros_like(s_ref)
        c_ref[...] = jnp.zeros_like(c_ref)

    rows = r * R + jax.lax.broadcasted_iota(jnp.int32, (R, 1), 0)
    cnt_le = jnp.sum((sl_ref[...] <= rows).astype(jnp.int32),
                     axis=1, keepdims=True)
    seg = jnp.clip(cnt_le, 0, B - 1)
    valid = (rows < N2).astype(jnp.float32)
    onehot = (seg == jax.lax.broadcasted_iota(jnp.int32, (R, B), 1)
              ).astype(jnp.float32) * valid
    s_ref[...] += jnp.dot(onehot.T, h_ref[...],
                          preferred_element_type=jnp.float32)
    c_ref[...] += jnp.sum(onehot, axis=0, keepdims=True)

    @pl.when(r == nblk - 1)
    def _():
        mean = s_ref[...] / jnp.maximum(c_ref[...], 1.0).T
        logits = jnp.dot(mean, fw_ref[...],
                         preferred_element_type=jnp.float32) + fb_ref[...]
        m = jnp.max(logits, axis=1, keepdims=True)
        lse = m + jnp.log(jnp.sum(jnp.exp(logits - m), axis=1, keepdims=True))
        o_ref[...] = logits - lse


def _readout(h, slice_, fc_w, fc_b):
    n = h.shape[0]
    grid = n // R
    out, _, _ = pl.pallas_call(
        _readout_body,
        grid=(grid,),
        in_specs=[
            pl.BlockSpec((R, 512), lambda r: (r, 0)),
            pl.BlockSpec((1, B), lambda r: (0, 0)),
            pl.BlockSpec((512, 10), lambda r: (0, 0)),
            pl.BlockSpec((1, 10), lambda r: (0, 0)),
        ],
        out_specs=[
            pl.BlockSpec((B, 10), lambda r: (0, 0)),
            pl.BlockSpec((B, 512), lambda r: (0, 0)),
            pl.BlockSpec((1, B), lambda r: (0, 0)),
        ],
        out_shape=[
            jax.ShapeDtypeStruct((B, 10), jnp.float32),
            jax.ShapeDtypeStruct((B, 512), jnp.float32),
            jax.ShapeDtypeStruct((1, B), jnp.float32),
        ],
    )(h, slice_.reshape(1, B).astype(jnp.int32), fc_w,
      fc_b.reshape(1, 10))
    return out


def kernel(x, edge_index0, pseudo0, edge_index1, pseudo1, edge_index2,
           pseudo2, slice_, W1, root1, b1, W2, root2, b2, W3, root3, b3,
           fc_w, fc_b):
    x0 = jnp.zeros((NP0, 2), jnp.float32).at[:N0].set(x)
    S0, deg0 = _scatter_S(x0, edge_index0, pseudo0, NP0, 1)
    h = _dense_layer(S0, x0, deg0, W1, root1, b1, 1)
    h = _pool(h)

    S1, deg1 = _scatter_S(h, edge_index1, pseudo1, NP1, 2)
    h = _dense_layer(S1, h, deg1, W2, root2, b2, 2)
    h = _pool(h)

    S2, deg2 = _scatter_S(h, edge_index2, pseudo2, NP2, 4)
    h = _dense_layer(S2, h, deg2, W3, root3, b3, 4)

    return _readout(h, slice_, fc_w, fc_b)
